# Initial kernel scaffold; baseline (speedup 1.0000x reference)
import jax
import jax.numpy as jnp
from jax.experimental import pallas as pl


def kernel(x, pos, W1, b1, gn1_g, gn1_b, dw_w, dwbn_g, dwbn_b, W2, b2, gn2_g, gn2_b):
    raise NotImplementedError("write your pallas kernel here")


# speedup vs baseline: 11.0713x; 11.0713x over previous
import functools

import jax
import jax.numpy as jnp
from jax import lax
from jax.experimental import pallas as pl
from jax.experimental.pallas import tpu as pltpu
from jax.experimental.pallas import tpu_sc as plsc

_RADIUS = 0.12
_K = 32
_EPS = 1e-5
_INV_SQRT2 = 0.7071067811865476


def _gelu(v):
    return 0.5 * v * (1.0 + lax.erf(v * _INV_SQRT2))


def _k1_body(x_ref, w1_ref, b1_ref, g1_ref, be1_ref, dw_ref, out_ref):
    x = x_ref[0]
    w1 = w1_ref[...]
    z = lax.dot_general(x, w1, (((0,), (1,)), ((), ())),
                        preferred_element_type=jnp.float32)
    z = z + b1_ref[...]
    n, mid = z.shape
    gw = mid // 4
    cnt = float(n * gw)
    blks = []
    for g in range(4):
        blk = z[:, g * gw:(g + 1) * gw]
        mu = jnp.sum(blk) / cnt
        var = jnp.sum(blk * blk) / cnt - mu * mu
        blks.append((blk - mu) * lax.rsqrt(var + _EPS))
    zn = jnp.concatenate(blks, axis=1)
    h = _gelu(zn * g1_ref[...] + be1_ref[...])
    out_ref[0] = h * dw_ref[...]


def _k1(x, w1, b1, g1, be1, dw):
    b, c, n = x.shape
    mid = w1.shape[0]
    vec = lambda a: a.reshape(1, -1)
    return pl.pallas_call(
        _k1_body,
        grid=(b,),
        in_specs=[
            pl.BlockSpec((1, c, n), lambda i: (i, 0, 0)),
            pl.BlockSpec((mid, c), lambda i: (0, 0)),
            pl.BlockSpec((1, mid), lambda i: (0, 0)),
            pl.BlockSpec((1, mid), lambda i: (0, 0)),
            pl.BlockSpec((1, mid), lambda i: (0, 0)),
            pl.BlockSpec((1, mid), lambda i: (0, 0)),
        ],
        out_specs=pl.BlockSpec((1, n, mid), lambda i: (i, 0, 0)),
        out_shape=jax.ShapeDtypeStruct((b, n, mid), jnp.float32),
    )(x, w1, vec(b1), vec(g1), vec(be1), vec(dw))


_ROWT = 256


def _k2_body(posf_ref, post_ref, idx_ref, w_ref):
    b = pl.program_id(0)
    i = pl.program_id(1)
    pc = posf_ref[0]
    pr = post_ref[0]
    n = pc.shape[1]
    d = lax.dot_general(pr.astype(jnp.bfloat16), pc.astype(jnp.bfloat16),
                        (((1,), (0,)), ((), ())),
                        preferred_element_type=jnp.float32)
    ssc = ((pc[0:1, :] * pc[0:1, :] + pc[1:2, :] * pc[1:2, :])
           + pc[2:3, :] * pc[2:3, :])
    ssr = ((pr[:, 0:1] * pr[:, 0:1] + pr[:, 1:2] * pr[:, 1:2])
           + pr[:, 2:3] * pr[:, 2:3])
    sqr = (-2.0 * d + ssr) + ssc
    sel = sqr <= jnp.float32(_RADIUS * _RADIUS)
    rank = sel.astype(jnp.int32)
    sh = 1
    while sh < n:
        rank = rank + jnp.concatenate(
            [jnp.zeros((_ROWT, sh), jnp.int32), rank[:, :n - sh]], axis=1)
        sh *= 2
    total = rank[:, n - 1:n]
    padcnt = (_K - jnp.minimum(total, _K)).astype(jnp.float32)
    sel32 = sel & (rank <= _K)
    first_ind = sel & (rank == 1)
    wrow = jnp.sum(sel32.astype(jnp.float32)
                   + padcnt * first_ind.astype(jnp.float32),
                   axis=0, keepdims=True)

    @pl.when(i == 0)
    def _():
        w_ref[0] = wrow

    @pl.when(i > 0)
    def _():
        w_ref[0] = w_ref[0] + wrow

    iota = lax.broadcasted_iota(jnp.int32, (_ROWT, n), 1)
    iosel = jnp.where(sel, iota + 1, 0)
    cols = []
    first = None
    for k in range(_K):
        s = jnp.sum(jnp.where(rank == (k + 1), iosel, 0),
                    axis=1, keepdims=True)
        if k == 0:
            first = s - 1
            cols.append(first)
        else:
            cols.append(jnp.where(s > 0, s - 1, first))
    idx_ref[0] = jnp.concatenate(cols, axis=1) + b * n


def _k2(pos):
    b, _, n = pos.shape
    nt = n // _ROWT
    pos_t = jnp.transpose(pos, (0, 2, 1))
    return pl.pallas_call(
        _k2_body,
        grid=(b, nt),
        in_specs=[
            pl.BlockSpec((1, 3, n), lambda bb, ii: (bb, 0, 0)),
            pl.BlockSpec((1, _ROWT, 3), lambda bb, ii: (bb, ii, 0)),
        ],
        out_specs=(
            pl.BlockSpec((1, _ROWT, _K), lambda bb, ii: (bb, ii, 0)),
            pl.BlockSpec((1, 1, n), lambda bb, ii: (bb, 0, 0)),
        ),
        out_shape=(
            jax.ShapeDtypeStruct((b, n, _K), jnp.int32),
            jax.ShapeDtypeStruct((b, 1, n), jnp.float32),
        ),
    )(pos, pos_t)


def _gather_max(table, idxf):
    tot, mid = table.shape
    info = plsc.get_sparse_core_info()
    nc, ns, lanes = info.num_cores, info.num_subcores, info.num_lanes
    nw = nc * ns
    ppw = tot // nw
    ch = 4
    nlg = mid // lanes
    mesh = plsc.VectorSubcoreMesh(core_axis_name="c", subcore_axis_name="s")

    @functools.partial(
        pl.kernel, mesh=mesh,
        out_type=jax.ShapeDtypeStruct((tot, mid), jnp.float32),
        scratch_types=[
            pltpu.VMEM((ppw * _K,), jnp.int32),
            pltpu.VMEM((ch * _K, mid), jnp.float32),
            pltpu.VMEM((ch, mid), jnp.float32),
            pltpu.SemaphoreType.DMA,
        ],
    )
    def sc_k(table_hbm, idx_hbm, out_hbm, idx_v, rows_v, out_v, sem):
        wid = lax.axis_index("s") * nc + lax.axis_index("c")
        base = wid * ppw
        pltpu.sync_copy(idx_hbm.at[pl.ds(base * _K, ppw * _K)], idx_v)

        def chunk_body(ci, carry):
            pltpu.async_copy(
                table_hbm.at[idx_v.at[pl.ds(ci * (ch * _K), ch * _K)]],
                rows_v, sem).wait()
            for p in range(ch):
                accs = tuple(rows_v[p * _K, pl.ds(l * lanes, lanes)]
                             for l in range(nlg))

                def red(kk, a):
                    return tuple(
                        jnp.maximum(a[l],
                                    rows_v[p * _K + kk, pl.ds(l * lanes, lanes)])
                        for l in range(nlg))

                accs = lax.fori_loop(1, _K, red, accs)
                for l in range(nlg):
                    out_v[p, pl.ds(l * lanes, lanes)] = accs[l]
            pltpu.sync_copy(out_v, out_hbm.at[pl.ds(base + ci * ch, ch)])
            return carry

        lax.fori_loop(0, ppw // ch, chunk_body, 0)

    return sc_k(table, idxf)


def _k4_body(s_ref, hts_ref, w_ref, x_ref, w2_ref, b2_ref, dg_ref, db_ref,
             g2_ref, be2_ref, out_ref):
    s = s_ref[0]
    hts = hts_ref[0]
    w = w_ref[0]
    n, mid = s.shape
    gw = mid // 4
    t1 = lax.dot_general(w, hts, (((1,), (0,)), ((), ())),
                         preferred_element_type=jnp.float32)
    t2 = lax.dot_general(w, hts * hts, (((1,), (0,)), ((), ())),
                         preferred_element_type=jnp.float32)
    denom = float(gw * n * _K)
    lane = lax.broadcasted_iota(jnp.int32, (1, mid), 1) // gw
    meanvec = jnp.zeros((1, mid), jnp.float32)
    invvec = jnp.zeros((1, mid), jnp.float32)
    for g in range(4):
        m = jnp.sum(t1[:, g * gw:(g + 1) * gw]) / denom
        q = jnp.sum(t2[:, g * gw:(g + 1) * gw]) / denom
        inv = lax.rsqrt(q - m * m + _EPS)
        meanvec = jnp.where(lane == g, m, meanvec)
        invvec = jnp.where(lane == g, inv, invvec)
    u = (s - meanvec) * invvec * dg_ref[...] + db_ref[...]
    y = _gelu(u)
    o = lax.dot_general(w2_ref[...], y, (((1,), (1,)), ((), ())),
                        preferred_element_type=jnp.float32)
    o = o + b2_ref[...]
    c = o.shape[0]
    gh = c // 4
    cnt = float(gh * n)
    blks = []
    for g in range(4):
        blk = o[g * gh:(g + 1) * gh, :]
        mu = jnp.sum(blk) / cnt
        var = jnp.sum(blk * blk) / cnt - mu * mu
        blks.append((blk - mu) * lax.rsqrt(var + _EPS))
    on = jnp.concatenate(blks, axis=0)
    out_ref[0] = on * g2_ref[...] + be2_ref[...] + x_ref[0]


def _k4(s, hts, w, x, w2, b2, dg, db, g2, be2):
    b, n, mid = s.shape
    c = x.shape[1]
    col = lambda a: a.reshape(-1, 1)
    vec = lambda a: a.reshape(1, -1)
    return pl.pallas_call(
        _k4_body,
        grid=(b,),
        in_specs=[
            pl.BlockSpec((1, n, mid), lambda i: (i, 0, 0)),
            pl.BlockSpec((1, n, mid), lambda i: (i, 0, 0)),
            pl.BlockSpec((1, 1, n), lambda i: (i, 0, 0)),
            pl.BlockSpec((1, c, n), lambda i: (i, 0, 0)),
            pl.BlockSpec((c, mid), lambda i: (0, 0)),
            pl.BlockSpec((c, 1), lambda i: (0, 0)),
            pl.BlockSpec((1, mid), lambda i: (0, 0)),
            pl.BlockSpec((1, mid), lambda i: (0, 0)),
            pl.BlockSpec((c, 1), lambda i: (0, 0)),
            pl.BlockSpec((c, 1), lambda i: (0, 0)),
        ],
        out_specs=pl.BlockSpec((1, c, n), lambda i: (i, 0, 0)),
        out_shape=jax.ShapeDtypeStruct((b, c, n), jnp.float32),
    )(s, hts, w, x, w2, col(b2), vec(dg), vec(db), col(g2), col(be2))


def kernel(x, pos, W1, b1, gn1_g, gn1_b, dw_w, dwbn_g, dwbn_b, W2, b2,
           gn2_g, gn2_b):
    b, c, n = x.shape
    mid = W1.shape[0]
    hts = _k1(x, W1, b1, gn1_g, gn1_b, dw_w)
    idx, w = _k2(pos)
    s = _gather_max(hts.reshape(b * n, mid),
                    idx.reshape(b * n * _K))
    s = s.reshape(b, n, mid)
    return _k4(s, hts, w, x, W2, b2, dwbn_g, dwbn_b, gn2_g, gn2_b)


# speedup vs baseline: 11.2181x; 1.0133x over previous
import functools

import jax
import jax.numpy as jnp
from jax import lax
from jax.experimental import pallas as pl
from jax.experimental.pallas import tpu as pltpu
from jax.experimental.pallas import tpu_sc as plsc

_RADIUS = 0.12
_K = 32
_EPS = 1e-5
_INV_SQRT2 = 0.7071067811865476


def _gelu(v):
    return 0.5 * v * (1.0 + lax.erf(v * _INV_SQRT2))


def _k1_body(x_ref, w1_ref, b1_ref, g1_ref, be1_ref, dw_ref, out_ref):
    x = x_ref[0]
    w1 = w1_ref[...]
    z = lax.dot_general(x, w1, (((0,), (1,)), ((), ())),
                        preferred_element_type=jnp.float32)
    z = z + b1_ref[...]
    n, mid = z.shape
    gw = mid // 4
    cnt = float(n * gw)
    blks = []
    for g in range(4):
        blk = z[:, g * gw:(g + 1) * gw]
        mu = jnp.sum(blk) / cnt
        var = jnp.sum(blk * blk) / cnt - mu * mu
        blks.append((blk - mu) * lax.rsqrt(var + _EPS))
    zn = jnp.concatenate(blks, axis=1)
    h = _gelu(zn * g1_ref[...] + be1_ref[...])
    out_ref[0] = h * dw_ref[...]


def _k1(x, w1, b1, g1, be1, dw):
    b, c, n = x.shape
    mid = w1.shape[0]
    vec = lambda a: a.reshape(1, -1)
    return pl.pallas_call(
        _k1_body,
        grid=(b,),
        in_specs=[
            pl.BlockSpec((1, c, n), lambda i: (i, 0, 0)),
            pl.BlockSpec((mid, c), lambda i: (0, 0)),
            pl.BlockSpec((1, mid), lambda i: (0, 0)),
            pl.BlockSpec((1, mid), lambda i: (0, 0)),
            pl.BlockSpec((1, mid), lambda i: (0, 0)),
            pl.BlockSpec((1, mid), lambda i: (0, 0)),
        ],
        out_specs=pl.BlockSpec((1, n, mid), lambda i: (i, 0, 0)),
        out_shape=jax.ShapeDtypeStruct((b, n, mid), jnp.float32),
    )(x, w1, vec(b1), vec(g1), vec(be1), vec(dw))


_ROWT = 256


def _k2_body(posf_ref, post_ref, idx_ref, w_ref):
    b = pl.program_id(0)
    i = pl.program_id(1)
    pc = posf_ref[0]
    pr = post_ref[0]
    n = pc.shape[1]
    d = lax.dot_general(pr.astype(jnp.bfloat16), pc.astype(jnp.bfloat16),
                        (((1,), (0,)), ((), ())),
                        preferred_element_type=jnp.float32)
    ssc = ((pc[0:1, :] * pc[0:1, :] + pc[1:2, :] * pc[1:2, :])
           + pc[2:3, :] * pc[2:3, :])
    ssr = ((pr[:, 0:1] * pr[:, 0:1] + pr[:, 1:2] * pr[:, 1:2])
           + pr[:, 2:3] * pr[:, 2:3])
    sqr = (-2.0 * d + ssr) + ssc
    sel = sqr <= jnp.float32(_RADIUS * _RADIUS)
    rank = sel.astype(jnp.int32)
    sh = 1
    while sh < n:
        rank = rank + jnp.concatenate(
            [jnp.zeros((_ROWT, sh), jnp.int32), rank[:, :n - sh]], axis=1)
        sh *= 2
    total = rank[:, n - 1:n]
    padcnt = (_K - jnp.minimum(total, _K)).astype(jnp.float32)
    sel32 = sel & (rank <= _K)
    first_ind = sel & (rank == 1)
    wrow = jnp.sum(sel32.astype(jnp.float32)
                   + padcnt * first_ind.astype(jnp.float32),
                   axis=0, keepdims=True)

    @pl.when(i == 0)
    def _():
        w_ref[0] = wrow

    @pl.when(i > 0)
    def _():
        w_ref[0] = w_ref[0] + wrow

    cols = []
    first = None
    for k in range(_K):
        s = jnp.sum((rank <= k).astype(jnp.int32),
                    axis=1, keepdims=True)
        if k == 0:
            first = s
            cols.append(s)
        else:
            cols.append(jnp.where(s < n, s, first))
    idx_ref[0] = jnp.concatenate(cols, axis=1) + b * n


def _k2(pos):
    b, _, n = pos.shape
    nt = n // _ROWT
    pos_t = jnp.transpose(pos, (0, 2, 1))
    return pl.pallas_call(
        _k2_body,
        grid=(b, nt),
        in_specs=[
            pl.BlockSpec((1, 3, n), lambda bb, ii: (bb, 0, 0)),
            pl.BlockSpec((1, _ROWT, 3), lambda bb, ii: (bb, ii, 0)),
        ],
        out_specs=(
            pl.BlockSpec((1, _ROWT, _K), lambda bb, ii: (bb, ii, 0)),
            pl.BlockSpec((1, 1, n), lambda bb, ii: (bb, 0, 0)),
        ),
        out_shape=(
            jax.ShapeDtypeStruct((b, n, _K), jnp.int32),
            jax.ShapeDtypeStruct((b, 1, n), jnp.float32),
        ),
    )(pos, pos_t)


def _gather_max(table, idxf):
    tot, mid = table.shape
    info = plsc.get_sparse_core_info()
    nc, ns, lanes = info.num_cores, info.num_subcores, info.num_lanes
    nw = nc * ns
    ppw = tot // nw
    ch = 4
    nlg = mid // lanes
    mesh = plsc.VectorSubcoreMesh(core_axis_name="c", subcore_axis_name="s")

    @functools.partial(
        pl.kernel, mesh=mesh,
        out_type=jax.ShapeDtypeStruct((tot, mid), jnp.float32),
        scratch_types=[
            pltpu.VMEM((ppw * _K,), jnp.int32),
            pltpu.VMEM((ch * _K, mid), jnp.float32),
            pltpu.VMEM((ch, mid), jnp.float32),
            pltpu.SemaphoreType.DMA,
        ],
    )
    def sc_k(table_hbm, idx_hbm, out_hbm, idx_v, rows_v, out_v, sem):
        wid = lax.axis_index("s") * nc + lax.axis_index("c")
        base = wid * ppw
        pltpu.sync_copy(idx_hbm.at[pl.ds(base * _K, ppw * _K)], idx_v)

        def chunk_body(ci, carry):
            pltpu.async_copy(
                table_hbm.at[idx_v.at[pl.ds(ci * (ch * _K), ch * _K)]],
                rows_v, sem).wait()
            for p in range(ch):
                accs = tuple(rows_v[p * _K, pl.ds(l * lanes, lanes)]
                             for l in range(nlg))

                def red(kk, a):
                    return tuple(
                        jnp.maximum(a[l],
                                    rows_v[p * _K + kk, pl.ds(l * lanes, lanes)])
                        for l in range(nlg))

                accs = lax.fori_loop(1, _K, red, accs)
                for l in range(nlg):
                    out_v[p, pl.ds(l * lanes, lanes)] = accs[l]
            pltpu.sync_copy(out_v, out_hbm.at[pl.ds(base + ci * ch, ch)])
            return carry

        lax.fori_loop(0, ppw // ch, chunk_body, 0)

    return sc_k(table, idxf)


def _k4_body(s_ref, hts_ref, w_ref, x_ref, w2_ref, b2_ref, dg_ref, db_ref,
             g2_ref, be2_ref, out_ref):
    s = s_ref[0]
    hts = hts_ref[0]
    w = w_ref[0]
    n, mid = s.shape
    gw = mid // 4
    t1 = lax.dot_general(w, hts, (((1,), (0,)), ((), ())),
                         preferred_element_type=jnp.float32)
    t2 = lax.dot_general(w, hts * hts, (((1,), (0,)), ((), ())),
                         preferred_element_type=jnp.float32)
    denom = float(gw * n * _K)
    lane = lax.broadcasted_iota(jnp.int32, (1, mid), 1) // gw
    meanvec = jnp.zeros((1, mid), jnp.float32)
    invvec = jnp.zeros((1, mid), jnp.float32)
    for g in range(4):
        m = jnp.sum(t1[:, g * gw:(g + 1) * gw]) / denom
        q = jnp.sum(t2[:, g * gw:(g + 1) * gw]) / denom
        inv = lax.rsqrt(q - m * m + _EPS)
        meanvec = jnp.where(lane == g, m, meanvec)
        invvec = jnp.where(lane == g, inv, invvec)
    u = (s - meanvec) * invvec * dg_ref[...] + db_ref[...]
    y = _gelu(u)
    o = lax.dot_general(w2_ref[...], y, (((1,), (1,)), ((), ())),
                        preferred_element_type=jnp.float32)
    o = o + b2_ref[...]
    c = o.shape[0]
    gh = c // 4
    cnt = float(gh * n)
    blks = []
    for g in range(4):
        blk = o[g * gh:(g + 1) * gh, :]
        mu = jnp.sum(blk) / cnt
        var = jnp.sum(blk * blk) / cnt - mu * mu
        blks.append((blk - mu) * lax.rsqrt(var + _EPS))
    on = jnp.concatenate(blks, axis=0)
    out_ref[0] = on * g2_ref[...] + be2_ref[...] + x_ref[0]


def _k4(s, hts, w, x, w2, b2, dg, db, g2, be2):
    b, n, mid = s.shape
    c = x.shape[1]
    col = lambda a: a.reshape(-1, 1)
    vec = lambda a: a.reshape(1, -1)
    return pl.pallas_call(
        _k4_body,
        grid=(b,),
        in_specs=[
            pl.BlockSpec((1, n, mid), lambda i: (i, 0, 0)),
            pl.BlockSpec((1, n, mid), lambda i: (i, 0, 0)),
            pl.BlockSpec((1, 1, n), lambda i: (i, 0, 0)),
            pl.BlockSpec((1, c, n), lambda i: (i, 0, 0)),
            pl.BlockSpec((c, mid), lambda i: (0, 0)),
            pl.BlockSpec((c, 1), lambda i: (0, 0)),
            pl.BlockSpec((1, mid), lambda i: (0, 0)),
            pl.BlockSpec((1, mid), lambda i: (0, 0)),
            pl.BlockSpec((c, 1), lambda i: (0, 0)),
            pl.BlockSpec((c, 1), lambda i: (0, 0)),
        ],
        out_specs=pl.BlockSpec((1, c, n), lambda i: (i, 0, 0)),
        out_shape=jax.ShapeDtypeStruct((b, c, n), jnp.float32),
    )(s, hts, w, x, w2, col(b2), vec(dg), vec(db), col(g2), col(be2))


def kernel(x, pos, W1, b1, gn1_g, gn1_b, dw_w, dwbn_g, dwbn_b, W2, b2,
           gn2_g, gn2_b):
    b, c, n = x.shape
    mid = W1.shape[0]
    hts = _k1(x, W1, b1, gn1_g, gn1_b, dw_w)
    idx, w = _k2(pos)
    s = _gather_max(hts.reshape(b * n, mid),
                    idx.reshape(b * n * _K))
    s = s.reshape(b, n, mid)
    return _k4(s, hts, w, x, W2, b2, dwbn_g, dwbn_b, gn2_g, gn2_b)


# speedup vs baseline: 13.0690x; 1.1650x over previous
import functools

import jax
import jax.numpy as jnp
from jax import lax
from jax.experimental import pallas as pl
from jax.experimental.pallas import tpu as pltpu
from jax.experimental.pallas import tpu_sc as plsc

_RADIUS = 0.12
_K = 32
_EPS = 1e-5
_INV_SQRT2 = 0.7071067811865476


def _gelu(v):
    return 0.5 * v * (1.0 + lax.erf(v * _INV_SQRT2))


def _k1_body(x_ref, w1_ref, b1_ref, g1_ref, be1_ref, dw_ref, out_ref):
    x = x_ref[0]
    w1 = w1_ref[...]
    z = lax.dot_general(x, w1, (((0,), (1,)), ((), ())),
                        preferred_element_type=jnp.float32)
    z = z + b1_ref[...]
    n, mid = z.shape
    gw = mid // 4
    cnt = float(n * gw)
    blks = []
    for g in range(4):
        blk = z[:, g * gw:(g + 1) * gw]
        mu = jnp.sum(blk) / cnt
        var = jnp.sum(blk * blk) / cnt - mu * mu
        blks.append((blk - mu) * lax.rsqrt(var + _EPS))
    zn = jnp.concatenate(blks, axis=1)
    h = _gelu(zn * g1_ref[...] + be1_ref[...])
    out_ref[0] = h * dw_ref[...]


def _k1(x, w1, b1, g1, be1, dw):
    b, c, n = x.shape
    mid = w1.shape[0]
    vec = lambda a: a.reshape(1, -1)
    return pl.pallas_call(
        _k1_body,
        grid=(b,),
        in_specs=[
            pl.BlockSpec((1, c, n), lambda i: (i, 0, 0)),
            pl.BlockSpec((mid, c), lambda i: (0, 0)),
            pl.BlockSpec((1, mid), lambda i: (0, 0)),
            pl.BlockSpec((1, mid), lambda i: (0, 0)),
            pl.BlockSpec((1, mid), lambda i: (0, 0)),
            pl.BlockSpec((1, mid), lambda i: (0, 0)),
        ],
        out_specs=pl.BlockSpec((1, n, mid), lambda i: (i, 0, 0)),
        out_shape=jax.ShapeDtypeStruct((b, n, mid), jnp.float32),
    )(x, w1, vec(b1), vec(g1), vec(be1), vec(dw))


_ROWT = 256


def _k2_body(posf_ref, post_ref, idx_ref, w_ref):
    b = pl.program_id(0)
    i = pl.program_id(1)
    pc = posf_ref[0]
    pr = post_ref[0]
    n = pc.shape[1]
    d = lax.dot_general(pr.astype(jnp.bfloat16), pc.astype(jnp.bfloat16),
                        (((1,), (0,)), ((), ())),
                        preferred_element_type=jnp.float32)
    ssc = ((pc[0:1, :] * pc[0:1, :] + pc[1:2, :] * pc[1:2, :])
           + pc[2:3, :] * pc[2:3, :])
    ssr = ((pr[:, 0:1] * pr[:, 0:1] + pr[:, 1:2] * pr[:, 1:2])
           + pr[:, 2:3] * pr[:, 2:3])
    sqr = (-2.0 * d + ssr) + ssc
    sel = sqr <= jnp.float32(_RADIUS * _RADIUS)
    rank = sel.astype(jnp.int32)
    sh = 1
    while sh < n:
        rank = rank + jnp.concatenate(
            [jnp.zeros((_ROWT, sh), jnp.int32), rank[:, :n - sh]], axis=1)
        sh *= 2
    total = rank[:, n - 1:n]
    padcnt = (_K - jnp.minimum(total, _K)).astype(jnp.float32)
    sel32 = sel & (rank <= _K)
    first_ind = sel & (rank == 1)
    wrow = jnp.sum(sel32.astype(jnp.float32)
                   + padcnt * first_ind.astype(jnp.float32),
                   axis=0, keepdims=True)

    @pl.when(i == 0)
    def _():
        w_ref[0] = wrow

    @pl.when(i > 0)
    def _():
        w_ref[0] = w_ref[0] + wrow

    cols = []
    first = None
    for k in range(_K):
        s = jnp.sum((rank <= k).astype(jnp.int32),
                    axis=1, keepdims=True)
        if k == 0:
            first = s
            cols.append(s)
        else:
            cols.append(jnp.where(s < n, s, first))
    idx_ref[0] = jnp.concatenate(cols, axis=1) + b * n


def _k2(pos):
    b, _, n = pos.shape
    nt = n // _ROWT
    pos_t = jnp.transpose(pos, (0, 2, 1))
    return pl.pallas_call(
        _k2_body,
        grid=(b, nt),
        in_specs=[
            pl.BlockSpec((1, 3, n), lambda bb, ii: (bb, 0, 0)),
            pl.BlockSpec((1, _ROWT, 3), lambda bb, ii: (bb, ii, 0)),
        ],
        out_specs=(
            pl.BlockSpec((1, _ROWT, _K), lambda bb, ii: (bb, ii, 0)),
            pl.BlockSpec((1, 1, n), lambda bb, ii: (bb, 0, 0)),
        ),
        out_shape=(
            jax.ShapeDtypeStruct((b, n, _K), jnp.int32),
            jax.ShapeDtypeStruct((b, 1, n), jnp.float32),
        ),
    )(pos, pos_t)


def _gather_max(table, idxf):
    tot, mid = table.shape
    info = plsc.get_sparse_core_info()
    nc, ns, lanes = info.num_cores, info.num_subcores, info.num_lanes
    nw = nc * ns
    ppw = tot // nw
    ch = 4
    nlg = mid // lanes
    mesh = plsc.VectorSubcoreMesh(core_axis_name="c", subcore_axis_name="s")

    nchunks = ppw // ch

    @functools.partial(
        pl.kernel, mesh=mesh,
        out_type=jax.ShapeDtypeStruct((tot, mid), jnp.float32),
        scratch_types=[
            pltpu.VMEM((ppw * _K,), jnp.int32),
            pltpu.VMEM((ch * _K, mid), jnp.float32),
            pltpu.VMEM((ch * _K, mid), jnp.float32),
            pltpu.VMEM((ch, mid), jnp.float32),
            pltpu.SemaphoreType.DMA,
            pltpu.SemaphoreType.DMA,
        ],
    )
    def sc_k(table_hbm, idx_hbm, out_hbm, idx_v, rows0, rows1, out_v,
             sem0, sem1):
        wid = lax.axis_index("s") * nc + lax.axis_index("c")
        base = wid * ppw
        pltpu.sync_copy(idx_hbm.at[pl.ds(base * _K, ppw * _K)], idx_v)
        bufs = ((rows0, sem0), (rows1, sem1))
        pltpu.async_copy(table_hbm.at[idx_v.at[pl.ds(0, ch * _K)]],
                         rows0, sem0)

        def pair_body(ci2, carry):
            for b in range(2):
                ci = ci2 * 2 + b
                rows_v, sem = bufs[b]
                nrows, nsem = bufs[1 - b]

                @pl.when(ci + 1 < nchunks)
                def _():
                    pltpu.async_copy(
                        table_hbm.at[idx_v.at[pl.ds((ci + 1) * (ch * _K),
                                                    ch * _K)]],
                        nrows, nsem)

                pltpu.make_async_copy(
                    table_hbm.at[idx_v.at[pl.ds(ci * (ch * _K), ch * _K)]],
                    rows_v, sem).wait()
                for p in range(ch):
                    accs = tuple(rows_v[p * _K, pl.ds(l * lanes, lanes)]
                                 for l in range(nlg))

                    def red(kk, a):
                        return tuple(
                            jnp.maximum(
                                a[l],
                                rows_v[p * _K + kk, pl.ds(l * lanes, lanes)])
                            for l in range(nlg))

                    accs = lax.fori_loop(1, _K, red, accs)
                    for l in range(nlg):
                        out_v[p, pl.ds(l * lanes, lanes)] = accs[l]
                pltpu.sync_copy(out_v, out_hbm.at[pl.ds(base + ci * ch, ch)])
            return carry

        lax.fori_loop(0, nchunks // 2, pair_body, 0)

    return sc_k(table, idxf)


def _k4_body(s_ref, hts_ref, w_ref, x_ref, w2_ref, b2_ref, dg_ref, db_ref,
             g2_ref, be2_ref, out_ref):
    s = s_ref[0]
    hts = hts_ref[0]
    w = w_ref[0]
    n, mid = s.shape
    gw = mid // 4
    t1 = lax.dot_general(w, hts, (((1,), (0,)), ((), ())),
                         preferred_element_type=jnp.float32)
    t2 = lax.dot_general(w, hts * hts, (((1,), (0,)), ((), ())),
                         preferred_element_type=jnp.float32)
    denom = float(gw * n * _K)
    lane = lax.broadcasted_iota(jnp.int32, (1, mid), 1) // gw
    meanvec = jnp.zeros((1, mid), jnp.float32)
    invvec = jnp.zeros((1, mid), jnp.float32)
    for g in range(4):
        m = jnp.sum(t1[:, g * gw:(g + 1) * gw]) / denom
        q = jnp.sum(t2[:, g * gw:(g + 1) * gw]) / denom
        inv = lax.rsqrt(q - m * m + _EPS)
        meanvec = jnp.where(lane == g, m, meanvec)
        invvec = jnp.where(lane == g, inv, invvec)
    u = (s - meanvec) * invvec * dg_ref[...] + db_ref[...]
    y = _gelu(u)
    o = lax.dot_general(w2_ref[...], y, (((1,), (1,)), ((), ())),
                        preferred_element_type=jnp.float32)
    o = o + b2_ref[...]
    c = o.shape[0]
    gh = c // 4
    cnt = float(gh * n)
    blks = []
    for g in range(4):
        blk = o[g * gh:(g + 1) * gh, :]
        mu = jnp.sum(blk) / cnt
        var = jnp.sum(blk * blk) / cnt - mu * mu
        blks.append((blk - mu) * lax.rsqrt(var + _EPS))
    on = jnp.concatenate(blks, axis=0)
    out_ref[0] = on * g2_ref[...] + be2_ref[...] + x_ref[0]


def _k4(s, hts, w, x, w2, b2, dg, db, g2, be2):
    b, n, mid = s.shape
    c = x.shape[1]
    col = lambda a: a.reshape(-1, 1)
    vec = lambda a: a.reshape(1, -1)
    return pl.pallas_call(
        _k4_body,
        grid=(b,),
        in_specs=[
            pl.BlockSpec((1, n, mid), lambda i: (i, 0, 0)),
            pl.BlockSpec((1, n, mid), lambda i: (i, 0, 0)),
            pl.BlockSpec((1, 1, n), lambda i: (i, 0, 0)),
            pl.BlockSpec((1, c, n), lambda i: (i, 0, 0)),
            pl.BlockSpec((c, mid), lambda i: (0, 0)),
            pl.BlockSpec((c, 1), lambda i: (0, 0)),
            pl.BlockSpec((1, mid), lambda i: (0, 0)),
            pl.BlockSpec((1, mid), lambda i: (0, 0)),
            pl.BlockSpec((c, 1), lambda i: (0, 0)),
            pl.BlockSpec((c, 1), lambda i: (0, 0)),
        ],
        out_specs=pl.BlockSpec((1, c, n), lambda i: (i, 0, 0)),
        out_shape=jax.ShapeDtypeStruct((b, c, n), jnp.float32),
    )(s, hts, w, x, w2, col(b2), vec(dg), vec(db), col(g2), col(be2))


def kernel(x, pos, W1, b1, gn1_g, gn1_b, dw_w, dwbn_g, dwbn_b, W2, b2,
           gn2_g, gn2_b):
    b, c, n = x.shape
    mid = W1.shape[0]
    hts = _k1(x, W1, b1, gn1_g, gn1_b, dw_w)
    idx, w = _k2(pos)
    s = _gather_max(hts.reshape(b * n, mid),
                    idx.reshape(b * n * _K))
    s = s.reshape(b, n, mid)
    return _k4(s, hts, w, x, W2, b2, dwbn_g, dwbn_b, gn2_g, gn2_b)
